# baseline (speedup 1.0000x reference)
import functools

import jax
import jax.numpy as jnp
from jax import lax
from jax.experimental import pallas as pl
from jax.experimental.pallas import tpu as pltpu
from jax.experimental.pallas import tpu_sc as plsc

_CLASS_NUM = 100000
_DIM = 128
_BATCH = 16384

_NC = 2
_NS = 16
_NW = _NC * _NS
_BPW = _BATCH // _NW
_CHUNK = 128
_NCHUNK = _BPW // _CHUNK
_LANES = 16
_M = 0.9


def _sc_update(keys, index, que, out_ref):
  mesh = plsc.VectorSubcoreMesh(core_axis_name="c", subcore_axis_name="s")

  @functools.partial(
      pl.kernel,
      mesh=mesh,
      out_type=(),
      scratch_types=[
          pltpu.VMEM((_NCHUNK, _CHUNK), jnp.int32),
          pltpu.VMEM((_BPW, _DIM), jnp.float32),
          pltpu.VMEM((_DIM,), jnp.float32),
          [pltpu.SemaphoreType.DMA] * _NCHUNK,
          [pltpu.SemaphoreType.DMA] * 1,
          pltpu.SemaphoreType.DMA,
          pltpu.SemaphoreType.DMA,
      ],
  )
  def k(keys_hbm, idx_hbm, que_hbm, out_hbm, idx_v, rows_v, keys_v, gsems,
        isems, ksem, ssem):
    wid = lax.axis_index("s") * _NC + lax.axis_index("c")
    base = wid * _BPW

    kcopy = pltpu.async_copy(keys_hbm, keys_v, ksem)
    icopy = pltpu.async_copy(
        idx_hbm.at[pl.ds(wid * _NCHUNK, _NCHUNK)], idx_v, isems[0]
    )

    icopy.wait()
    gathers = [
        pltpu.async_copy(
            que_hbm.at[idx_v.at[j]],
            rows_v.at[pl.ds(j * _CHUNK, _CHUNK)],
            gsems[j],
        )
        for j in range(_NCHUNK)
    ]

    kcopy.wait()
    kc = [keys_v[pl.ds(c * _LANES, _LANES)] * (1.0 - _M)
          for c in range(_DIM // _LANES)]

    scatters = []
    for j in range(_NCHUNK):
      gathers[j].wait()
      lo = j * _CHUNK

      def row_body(r):
        for c in range(_DIM // _LANES):
          sl = pl.ds(c * _LANES, _LANES)
          rows_v[r, sl] = rows_v[r, sl] * _M + kc[c]

      plsc.parallel_loop(lo, lo + _CHUNK, unroll=8)(row_body)

      scatters.append(
          pltpu.async_copy(
              rows_v.at[pl.ds(lo, _CHUNK)],
              out_hbm.at[idx_v.at[j]],
              ssem,
          )
      )
    for s in scatters:
      s.wait()

  k(keys, index, que, out_ref)


def kernel(keys, index, que):
  out_ref = jax.new_ref(que)
  idx2d = index.astype(jnp.int32).reshape(_BATCH // _CHUNK, _CHUNK)
  _sc_update(keys, idx2d, que, out_ref)
  return jax.freeze(out_ref)
